# baseline (speedup 1.0000x reference)
import functools

import jax
import jax.numpy as jnp
from jax import lax
from jax.experimental import pallas as pl
from jax.experimental.pallas import tpu as pltpu
from jax.experimental.pallas import tpu_sc as plsc

NC, NS, L = 2, 16, 16
B, C, H, W = 32, 7, 256, 256
N = 100
NPAD = 112
NCHUNK = NPAD // L


def _sc_gather_body(preds_hbm, boxes_hbm, out_hbm,
                    cx_v, cy_v, idx_v, lanes_v, cells_v, grid_v, rows_v,
                    vals_v, sem0, sem1, sem2, sem3):
    sems = [sem0, sem1, sem2, sem3]
    b = lax.axis_index("s") * NC + lax.axis_index("c")
    pltpu.sync_copy(boxes_hbm.at[0, b], cx_v)
    pltpu.sync_copy(boxes_hbm.at[1, b], cy_v)
    iota = lax.iota(jnp.int32, L)
    base_b = b * (C * H)
    for c in range(NCHUNK):
        cx = cx_v[pl.ds(c * L, L)]
        cy = cy_v[pl.ds(c * L, L)]
        ii = (cy * float(H)).astype(jnp.int32)
        jj = (cx * float(W)).astype(jnp.int32)
        lanes_v[pl.ds(c * L, L)] = jj
        cells_v[pl.ds(c * L, L)] = ii * W + jj
        base = base_b + ii
        for ch in range(C):
            idx_v[ch, pl.ds(c * L, L)] = base + ch * H
    segs = [(ch, base, ln) for ch in range(C) for base, ln in
            ((0, 32), (32, 32), (64, 32), (96, 8))]
    NBUF = 4

    def _fire(t):
        ch, base, ln = segs[t]
        return pltpu.async_copy(
            preds_hbm.at[idx_v.at[ch, pl.ds(base, ln)]],
            rows_v.at[t % NBUF, pl.ds(0, ln)], sems[t % NBUF])

    copies = [_fire(t) for t in range(NBUF)]
    for c in range(NCHUNK):
        n_vec = iota + c * L
        plsc.store_scatter(grid_v, [cells_v[pl.ds(c * L, L)]], n_vec,
                           mask=n_vec < N)
    for c in range(NCHUNK):
        n_vec = iota + c * L
        winner = plsc.load_gather(grid_v, [cells_v[pl.ds(c * L, L)]])
        valid = jnp.logical_and(winner == n_vec, n_vec < N)
        vals_v[C, pl.ds(c * L, L)] = jnp.where(valid, 1.0, 0.0)
    for t, (ch, base, ln) in enumerate(segs):
        copies[t % NBUF].wait()
        buf = rows_v.at[t % NBUF]
        cover = ln if ln % L == 0 else NPAD - base
        for c in range(cover // L):
            rowsel = jnp.minimum(iota + c * L, ln - 1)
            v = plsc.load_gather(
                buf, [rowsel, lanes_v[pl.ds(base + c * L, L)]])
            vals_v[ch, pl.ds(base + c * L, L)] = v
        if t + NBUF < len(segs):
            copies[t % NBUF] = _fire(t + NBUF)
    pltpu.sync_copy(vals_v, out_hbm.at[:, b])


def _sc_gather(preds_rows, boxes_c):
    mesh = plsc.VectorSubcoreMesh(core_axis_name="c", subcore_axis_name="s",
                                  num_cores=NC, num_subcores=NS)
    run = functools.partial(
        pl.kernel,
        out_type=jax.ShapeDtypeStruct((C + 1, B, NPAD), jnp.float32),
        mesh=mesh,
        compiler_params=pltpu.CompilerParams(needs_layout_passes=False,
                                             skip_device_barrier=True,
                                             disable_bounds_checks=True,
                                             disable_semaphore_checks=True),
        scratch_types=[
            pltpu.VMEM((NPAD,), jnp.float32),
            pltpu.VMEM((NPAD,), jnp.float32),
            pltpu.VMEM((C, NPAD), jnp.int32),
            pltpu.VMEM((NPAD,), jnp.int32),
            pltpu.VMEM((NPAD,), jnp.int32),
            pltpu.VMEM((H * W,), jnp.int32),
            pltpu.VMEM((4, 32, W), jnp.float32),
            pltpu.VMEM((C + 1, NPAD), jnp.float32),
            pltpu.SemaphoreType.DMA,
            pltpu.SemaphoreType.DMA,
            pltpu.SemaphoreType.DMA,
            pltpu.SemaphoreType.DMA,
        ],
    )(_sc_gather_body)
    return run(preds_rows, boxes_c)


def _softplus(x):
    return jnp.maximum(x, 0.0) + jnp.log(1.0 + jnp.exp(-jnp.abs(x)))


DB = 4


def _tc_dense_body(obj_ref, out_ref):
    step = pl.program_id(0)

    @pl.when(step == 0)
    def _():
        out_ref[0, 0] = 0.0

    x = obj_ref[:, 0]
    out_ref[0, 0] += jnp.sum(_softplus(x))


def _tc_dense(preds):
    return pl.pallas_call(
        _tc_dense_body,
        grid=(B // DB,),
        in_specs=[pl.BlockSpec((DB, 1, H, W), lambda i: (i, 4, 0, 0))],
        out_specs=pl.BlockSpec((1, 1), lambda i: (0, 0),
                               memory_space=pltpu.SMEM),
        out_shape=jax.ShapeDtypeStruct((1, 1), jnp.float32),
    )(preds)


def _tc_sparse_body(vals_ref, boxes_c_ref, lab_ref, dense_ref, out_ref):
    valid = vals_ref[C]
    box_l1 = jnp.zeros((), jnp.float32)
    for k in range(4):
        box_l1 += jnp.sum(jnp.abs(vals_ref[k] - boxes_c_ref[k]) * valid)
    pobj = jnp.sum(vals_ref[4] * valid)
    c0 = vals_ref[5]
    c1 = vals_ref[6]
    lf = lab_ref[...]
    c_sel = c0 * (1.0 - lf) + c1 * lf
    cls = jnp.sum((_softplus(c0) + _softplus(c1) - c_sel) * valid)
    total = dense_ref[0, 0] + 5.0 * box_l1 - pobj + cls
    out_ref[0, 0] = total * (1.0 / B)


def _tc_sparse(vals, boxes_c, lab, dense):
    return pl.pallas_call(
        _tc_sparse_body,
        in_specs=[
            pl.BlockSpec((C + 1, B, NPAD), lambda: (0, 0, 0)),
            pl.BlockSpec((4, B, NPAD), lambda: (0, 0, 0)),
            pl.BlockSpec((B, NPAD), lambda: (0, 0)),
            pl.BlockSpec((1, 1), lambda: (0, 0), memory_space=pltpu.SMEM),
        ],
        out_specs=pl.BlockSpec((1, 1), lambda: (0, 0),
                               memory_space=pltpu.SMEM),
        out_shape=jax.ShapeDtypeStruct((1, 1), jnp.float32),
    )(vals, boxes_c, lab, dense)


def kernel(preds, boxes, labels):
    preds_rows = preds.reshape(B * C * H, W)
    boxes_c = jnp.pad(jnp.transpose(boxes, (2, 0, 1)),
                      ((0, 0), (0, 0), (0, NPAD - N)))
    lab = jnp.pad(labels.astype(jnp.float32), ((0, 0), (0, NPAD - N)))
    vals = _sc_gather(preds_rows, boxes_c)
    dense = _tc_dense(preds)
    out = _tc_sparse(vals, boxes_c, lab, dense)
    return out[0, 0]


# speedup vs baseline: 1.0032x; 1.0032x over previous
import functools

import jax
import jax.numpy as jnp
from jax import lax
from jax.experimental import pallas as pl
from jax.experimental.pallas import tpu as pltpu
from jax.experimental.pallas import tpu_sc as plsc

NC, NS, L = 2, 16, 16
B, C, H, W = 32, 7, 256, 256
N = 100
NPAD = 112
NCHUNK = NPAD // L


def _sc_gather_body(preds_hbm, boxes_hbm, out_hbm,
                    cx_v, cy_v, idx_v, lanes_v, cells_v, grid_v, rows_v,
                    vals_v, sem0, sem1, sem2, sem3):
    sems = [sem0, sem1, sem2, sem3]
    b = lax.axis_index("s") * NC + lax.axis_index("c")
    pltpu.sync_copy(boxes_hbm.at[0, b], cx_v)
    pltpu.sync_copy(boxes_hbm.at[1, b], cy_v)
    iota = lax.iota(jnp.int32, L)
    base_b = b * (C * H)
    for c in range(NCHUNK):
        cx = cx_v[pl.ds(c * L, L)]
        cy = cy_v[pl.ds(c * L, L)]
        ii = (cy * float(H)).astype(jnp.int32)
        jj = (cx * float(W)).astype(jnp.int32)
        lanes_v[pl.ds(c * L, L)] = jj
        cells_v[pl.ds(c * L, L)] = ii * W + jj
        base = base_b + ii
        for ch in range(C):
            idx_v[ch, pl.ds(c * L, L)] = base + ch * H
    segs = [(ch, base, ln) for ch in range(C) for base, ln in
            ((0, 32), (32, 32), (64, 32), (96, 8))]
    NBUF = 4

    def _fire(t):
        ch, base, ln = segs[t]
        return pltpu.async_copy(
            preds_hbm.at[idx_v.at[ch, pl.ds(base, ln)]],
            rows_v.at[t % NBUF, pl.ds(0, ln)], sems[t % NBUF])

    copies = [_fire(t) for t in range(NBUF)]
    for c in range(NCHUNK):
        n_vec = iota + c * L
        plsc.store_scatter(grid_v, [cells_v[pl.ds(c * L, L)]], n_vec,
                           mask=n_vec < N)
    for c in range(NCHUNK):
        n_vec = iota + c * L
        winner = plsc.load_gather(grid_v, [cells_v[pl.ds(c * L, L)]])
        valid = jnp.logical_and(winner == n_vec, n_vec < N)
        vals_v[C, pl.ds(c * L, L)] = jnp.where(valid, 1.0, 0.0)
    for t, (ch, base, ln) in enumerate(segs):
        copies[t % NBUF].wait()
        buf = rows_v.at[t % NBUF]
        cover = ln if ln % L == 0 else NPAD - base
        for c in range(cover // L):
            rowsel = jnp.minimum(iota + c * L, ln - 1)
            v = plsc.load_gather(
                buf, [rowsel, lanes_v[pl.ds(base + c * L, L)]])
            vals_v[ch, pl.ds(base + c * L, L)] = v
        if t + NBUF < len(segs):
            copies[t % NBUF] = _fire(t + NBUF)
    pltpu.sync_copy(vals_v, out_hbm.at[:, b])


def _sc_gather(preds_rows, boxes_c):
    mesh = plsc.VectorSubcoreMesh(core_axis_name="c", subcore_axis_name="s",
                                  num_cores=NC, num_subcores=NS)
    run = functools.partial(
        pl.kernel,
        out_type=jax.ShapeDtypeStruct((C + 1, B, NPAD), jnp.float32),
        mesh=mesh,
        compiler_params=pltpu.CompilerParams(needs_layout_passes=False,
                                             skip_device_barrier=True),
        scratch_types=[
            pltpu.VMEM((NPAD,), jnp.float32),
            pltpu.VMEM((NPAD,), jnp.float32),
            pltpu.VMEM((C, NPAD), jnp.int32),
            pltpu.VMEM((NPAD,), jnp.int32),
            pltpu.VMEM((NPAD,), jnp.int32),
            pltpu.VMEM((H * W,), jnp.int32),
            pltpu.VMEM((4, 32, W), jnp.float32),
            pltpu.VMEM((C + 1, NPAD), jnp.float32),
            pltpu.SemaphoreType.DMA,
            pltpu.SemaphoreType.DMA,
            pltpu.SemaphoreType.DMA,
            pltpu.SemaphoreType.DMA,
        ],
    )(_sc_gather_body)
    return run(preds_rows, boxes_c)


def _softplus(x):
    return jnp.maximum(x, 0.0) + jnp.log(1.0 + jnp.exp(-jnp.abs(x)))


DB = 4


def _tc_dense_body(obj_ref, out_ref):
    step = pl.program_id(0)

    @pl.when(step == 0)
    def _():
        out_ref[0, 0] = 0.0

    x = obj_ref[:, 0]
    out_ref[0, 0] += jnp.sum(_softplus(x))


def _tc_dense(preds):
    return pl.pallas_call(
        _tc_dense_body,
        grid=(B // DB,),
        in_specs=[pl.BlockSpec((DB, 1, H, W), lambda i: (i, 4, 0, 0))],
        out_specs=pl.BlockSpec((1, 1), lambda i: (0, 0),
                               memory_space=pltpu.SMEM),
        out_shape=jax.ShapeDtypeStruct((1, 1), jnp.float32),
    )(preds)


def _tc_sparse_body(vals_ref, boxes_c_ref, lab_ref, dense_ref, out_ref):
    valid = vals_ref[C]
    box_l1 = jnp.zeros((), jnp.float32)
    for k in range(4):
        box_l1 += jnp.sum(jnp.abs(vals_ref[k] - boxes_c_ref[k]) * valid)
    pobj = jnp.sum(vals_ref[4] * valid)
    c0 = vals_ref[5]
    c1 = vals_ref[6]
    lf = lab_ref[...]
    c_sel = c0 * (1.0 - lf) + c1 * lf
    cls = jnp.sum((_softplus(c0) + _softplus(c1) - c_sel) * valid)
    total = dense_ref[0, 0] + 5.0 * box_l1 - pobj + cls
    out_ref[0, 0] = total * (1.0 / B)


def _tc_sparse(vals, boxes_c, lab, dense):
    return pl.pallas_call(
        _tc_sparse_body,
        in_specs=[
            pl.BlockSpec((C + 1, B, NPAD), lambda: (0, 0, 0)),
            pl.BlockSpec((4, B, NPAD), lambda: (0, 0, 0)),
            pl.BlockSpec((B, NPAD), lambda: (0, 0)),
            pl.BlockSpec((1, 1), lambda: (0, 0), memory_space=pltpu.SMEM),
        ],
        out_specs=pl.BlockSpec((1, 1), lambda: (0, 0),
                               memory_space=pltpu.SMEM),
        out_shape=jax.ShapeDtypeStruct((1, 1), jnp.float32),
    )(vals, boxes_c, lab, dense)


def kernel(preds, boxes, labels):
    preds_rows = preds.reshape(B * C * H, W)
    boxes_c = jnp.pad(jnp.transpose(boxes, (2, 0, 1)),
                      ((0, 0), (0, 0), (0, NPAD - N)))
    lab = jnp.pad(labels.astype(jnp.float32), ((0, 0), (0, NPAD - N)))
    vals = _sc_gather(preds_rows, boxes_c)
    dense = _tc_dense(preds)
    out = _tc_sparse(vals, boxes_c, lab, dense)
    return out[0, 0]


# speedup vs baseline: 1.0468x; 1.0435x over previous
import functools

import jax
import jax.numpy as jnp
from jax import lax
from jax.experimental import pallas as pl
from jax.experimental.pallas import tpu as pltpu
from jax.experimental.pallas import tpu_sc as plsc

NC, NS, L = 2, 16, 16
B, C, H, W = 32, 7, 256, 256
N = 100
NPAD = 112
NCHUNK = NPAD // L


def _sc_gather_body(preds_hbm, boxes_hbm, out_hbm,
                    cx_v, cy_v, idx_v, lanes_v, cells_v, grid_v, rows_v,
                    vals_v, sem0, sem1, sem2, sem3, sem4, sem5):
    sems = [sem0, sem1, sem2, sem3, sem4, sem5]
    b = lax.axis_index("s") * NC + lax.axis_index("c")
    pltpu.sync_copy(boxes_hbm.at[0, b], cx_v)
    pltpu.sync_copy(boxes_hbm.at[1, b], cy_v)
    iota = lax.iota(jnp.int32, L)
    base_b = b * (C * H)
    for c in range(NCHUNK):
        cx = cx_v[pl.ds(c * L, L)]
        cy = cy_v[pl.ds(c * L, L)]
        ii = (cy * float(H)).astype(jnp.int32)
        jj = (cx * float(W)).astype(jnp.int32)
        lanes_v[pl.ds(c * L, L)] = jj
        cells_v[pl.ds(c * L, L)] = ii * W + jj
        base = base_b + ii
        for ch in range(C):
            idx_v[ch, pl.ds(c * L, L)] = base + ch * H
    segs = [(ch, base, ln) for ch in range(C) for base, ln in
            ((0, 32), (32, 32), (64, 32), (96, 8))]
    NBUF = 6

    def _fire(t):
        ch, base, ln = segs[t]
        return pltpu.async_copy(
            preds_hbm.at[idx_v.at[ch, pl.ds(base, ln)]],
            rows_v.at[t % NBUF, pl.ds(0, ln)], sems[t % NBUF])

    copies = [_fire(t) for t in range(NBUF)]
    for c in range(NCHUNK):
        n_vec = iota + c * L
        plsc.store_scatter(grid_v, [cells_v[pl.ds(c * L, L)]], n_vec,
                           mask=n_vec < N)
    for c in range(NCHUNK):
        n_vec = iota + c * L
        winner = plsc.load_gather(grid_v, [cells_v[pl.ds(c * L, L)]])
        valid = jnp.logical_and(winner == n_vec, n_vec < N)
        vals_v[C, pl.ds(c * L, L)] = jnp.where(valid, 1.0, 0.0)
    for t, (ch, base, ln) in enumerate(segs):
        copies[t % NBUF].wait()
        buf = rows_v.at[t % NBUF]
        cover = ln if ln % L == 0 else NPAD - base
        for c in range(cover // L):
            rowsel = jnp.minimum(iota + c * L, ln - 1)
            v = plsc.load_gather(
                buf, [rowsel, lanes_v[pl.ds(base + c * L, L)]])
            vals_v[ch, pl.ds(base + c * L, L)] = v
        if t + NBUF < len(segs):
            copies[t % NBUF] = _fire(t + NBUF)
    pltpu.sync_copy(vals_v, out_hbm.at[:, b])


def _sc_gather(preds_rows, boxes_c):
    mesh = plsc.VectorSubcoreMesh(core_axis_name="c", subcore_axis_name="s",
                                  num_cores=NC, num_subcores=NS)
    run = functools.partial(
        pl.kernel,
        out_type=jax.ShapeDtypeStruct((C + 1, B, NPAD), jnp.float32),
        mesh=mesh,
        compiler_params=pltpu.CompilerParams(needs_layout_passes=False,
                                             skip_device_barrier=True),
        scratch_types=[
            pltpu.VMEM((NPAD,), jnp.float32),
            pltpu.VMEM((NPAD,), jnp.float32),
            pltpu.VMEM((C, NPAD), jnp.int32),
            pltpu.VMEM((NPAD,), jnp.int32),
            pltpu.VMEM((NPAD,), jnp.int32),
            pltpu.VMEM((H * W,), jnp.int32),
            pltpu.VMEM((6, 32, W), jnp.float32),
            pltpu.VMEM((C + 1, NPAD), jnp.float32),
            pltpu.SemaphoreType.DMA,
            pltpu.SemaphoreType.DMA,
            pltpu.SemaphoreType.DMA,
            pltpu.SemaphoreType.DMA,
            pltpu.SemaphoreType.DMA,
            pltpu.SemaphoreType.DMA,
        ],
    )(_sc_gather_body)
    return run(preds_rows, boxes_c)


def _softplus(x):
    return jnp.maximum(x, 0.0) + jnp.log(1.0 + jnp.exp(-jnp.abs(x)))


DB = 4


def _tc_dense_body(obj_ref, out_ref):
    step = pl.program_id(0)

    @pl.when(step == 0)
    def _():
        out_ref[0, 0] = 0.0

    x = obj_ref[:, 0]
    out_ref[0, 0] += jnp.sum(_softplus(x))


def _tc_dense(preds):
    return pl.pallas_call(
        _tc_dense_body,
        grid=(B // DB,),
        in_specs=[pl.BlockSpec((DB, 1, H, W), lambda i: (i, 4, 0, 0))],
        out_specs=pl.BlockSpec((1, 1), lambda i: (0, 0),
                               memory_space=pltpu.SMEM),
        out_shape=jax.ShapeDtypeStruct((1, 1), jnp.float32),
    )(preds)


def _tc_sparse_body(vals_ref, boxes_c_ref, lab_ref, dense_ref, out_ref):
    valid = vals_ref[C]
    box_l1 = jnp.zeros((), jnp.float32)
    for k in range(4):
        box_l1 += jnp.sum(jnp.abs(vals_ref[k] - boxes_c_ref[k]) * valid)
    pobj = jnp.sum(vals_ref[4] * valid)
    c0 = vals_ref[5]
    c1 = vals_ref[6]
    lf = lab_ref[...]
    c_sel = c0 * (1.0 - lf) + c1 * lf
    cls = jnp.sum((_softplus(c0) + _softplus(c1) - c_sel) * valid)
    total = dense_ref[0, 0] + 5.0 * box_l1 - pobj + cls
    out_ref[0, 0] = total * (1.0 / B)


def _tc_sparse(vals, boxes_c, lab, dense):
    return pl.pallas_call(
        _tc_sparse_body,
        in_specs=[
            pl.BlockSpec((C + 1, B, NPAD), lambda: (0, 0, 0)),
            pl.BlockSpec((4, B, NPAD), lambda: (0, 0, 0)),
            pl.BlockSpec((B, NPAD), lambda: (0, 0)),
            pl.BlockSpec((1, 1), lambda: (0, 0), memory_space=pltpu.SMEM),
        ],
        out_specs=pl.BlockSpec((1, 1), lambda: (0, 0),
                               memory_space=pltpu.SMEM),
        out_shape=jax.ShapeDtypeStruct((1, 1), jnp.float32),
    )(vals, boxes_c, lab, dense)


def kernel(preds, boxes, labels):
    preds_rows = preds.reshape(B * C * H, W)
    boxes_c = jnp.pad(jnp.transpose(boxes, (2, 0, 1)),
                      ((0, 0), (0, 0), (0, NPAD - N)))
    lab = jnp.pad(labels.astype(jnp.float32), ((0, 0), (0, NPAD - N)))
    vals = _sc_gather(preds_rows, boxes_c)
    dense = _tc_dense(preds)
    out = _tc_sparse(vals, boxes_c, lab, dense)
    return out[0, 0]
